# baseline (speedup 1.0000x reference)
import jax
import jax.numpy as jnp
from jax.experimental import pallas as pl
from jax.experimental.pallas import tpu as pltpu


_NCHUNKS = 32
_DEPTH = 12


def _copy_body(in_ref, out_ref, scr, in_sems, out_sems):
    n, k = _NCHUNKS, _DEPTH

    def in_copy(i):
        return pltpu.make_async_copy(in_ref.at[i], scr.at[i % k],
                                     in_sems.at[i % k])

    def out_copy(i):
        return pltpu.make_async_copy(scr.at[i % k], out_ref.at[i],
                                     out_sems.at[i % k])

    for i in range(min(k, n)):
        in_copy(i).start()
    for i in range(n):
        in_copy(i).wait()
        out_copy(i).start()
        j = i + k
        if j < n:
            out_copy(i).wait()
            in_copy(j).start()
    for i in range(max(0, n - k), n):
        out_copy(i).wait()


def kernel(sequence):
    b, t, d = sequence.shape
    rows = b * t
    chunk_rows = rows // _NCHUNKS
    chunked = sequence.reshape(_NCHUNKS, chunk_rows, d)
    out = pl.pallas_call(
        _copy_body,
        out_shape=jax.ShapeDtypeStruct(chunked.shape, chunked.dtype),
        in_specs=[pl.BlockSpec(memory_space=pl.ANY)],
        out_specs=pl.BlockSpec(memory_space=pl.ANY),
        scratch_shapes=[
            pltpu.VMEM((_DEPTH, chunk_rows, d), jnp.float32),
            pltpu.SemaphoreType.DMA((_DEPTH,)),
            pltpu.SemaphoreType.DMA((_DEPTH,)),
        ],
    )(chunked)
    return out.reshape(b, t, d)


# speedup vs baseline: 1.0338x; 1.0338x over previous
import jax
import jax.numpy as jnp
from jax.experimental import pallas as pl
from jax.experimental.pallas import tpu as pltpu


_NCHUNKS = 8
_DEPTH = 5


def _copy_body(in_ref, out_ref, scr, in_sems, out_sems):
    n, k = _NCHUNKS, _DEPTH

    def in_copy(i):
        return pltpu.make_async_copy(in_ref.at[i], scr.at[i % k],
                                     in_sems.at[i % k])

    def out_copy(i):
        return pltpu.make_async_copy(scr.at[i % k], out_ref.at[i],
                                     out_sems.at[i % k])

    for i in range(min(k, n)):
        in_copy(i).start()
    for i in range(n):
        in_copy(i).wait()
        out_copy(i).start()
        j = i + k
        if j < n:
            out_copy(i).wait()
            in_copy(j).start()
    for i in range(max(0, n - k), n):
        out_copy(i).wait()


def kernel(sequence):
    b, t, d = sequence.shape
    rows = b * t
    chunk_rows = rows // _NCHUNKS
    chunked = sequence.reshape(_NCHUNKS, chunk_rows, d)
    out = pl.pallas_call(
        _copy_body,
        out_shape=jax.ShapeDtypeStruct(chunked.shape, chunked.dtype),
        in_specs=[pl.BlockSpec(memory_space=pl.ANY)],
        out_specs=pl.BlockSpec(memory_space=pl.ANY),
        scratch_shapes=[
            pltpu.VMEM((_DEPTH, chunk_rows, d), jnp.float32),
            pltpu.SemaphoreType.DMA((_DEPTH,)),
            pltpu.SemaphoreType.DMA((_DEPTH,)),
        ],
    )(chunked)
    return out.reshape(b, t, d)
